# baseline (speedup 1.0000x reference)
import functools

import jax
import jax.numpy as jnp
from jax import lax
from jax.experimental import pallas as pl
from jax.experimental.pallas import tpu as pltpu
from jax.experimental.pallas import tpu_sc as plsc

N_TERMS = 100000
D = 64
B = 16384
L = 50
N_ROWS = B * L


_NORM_BLK = 2000


def _norm_body(x_ref, o_ref):
    x = x_ref[...]
    ss = jnp.sum(x * x, axis=1, keepdims=True)
    o_ref[...] = x * lax.rsqrt(jnp.maximum(ss, 1e-24))


def _normalize_table(table):
    return pl.pallas_call(
        _norm_body,
        grid=(N_TERMS // _NORM_BLK,),
        in_specs=[pl.BlockSpec((_NORM_BLK, D), lambda i: (i, 0))],
        out_specs=pl.BlockSpec((_NORM_BLK, D), lambda i: (i, 0)),
        out_shape=jax.ShapeDtypeStruct((N_TERMS, D), jnp.float32),
    )(table)


_NC = 2
_NS = 16
_NW = _NC * _NS
_NB0 = B // 128
_B0_W = _NB0 // _NW
_SLABS = L * _B0_W

_sc_mesh = plsc.VectorSubcoreMesh(core_axis_name="c", subcore_axis_name="s")


@functools.partial(
    pl.kernel,
    out_type=jax.ShapeDtypeStruct((L, D // 8, 128, 8, 128), jnp.float32),
    mesh=_sc_mesh,
    compiler_params=pltpu.CompilerParams(
        use_tc_tiling_on_sc=False, needs_layout_passes=False),
    scratch_types=[
        pltpu.VMEM((L, _B0_W, 128), jnp.int32),
        pltpu.VMEM((4, 128, D), jnp.float32),
        pltpu.VMEM((2, D, 128), jnp.float32),
        pltpu.SemaphoreType.DMA,
        pltpu.SemaphoreType.DMA,
        pltpu.SemaphoreType.DMA,
        pltpu.SemaphoreType.DMA,
        pltpu.SemaphoreType.DMA,
        pltpu.SemaphoreType.DMA,
    ],
)
def _gather_kernel(idx_hbm, tab_hbm, out_hbm, idx_v, rows_v, tr_v,
                   gsem0, gsem1, gsem2, gsem3, ssem0, ssem1):
    wid = lax.axis_index("s") * _NC + lax.axis_index("c")

    pltpu.sync_copy(idx_hbm.at[:, pl.ds(wid * _B0_W, _B0_W)], idx_v)

    gsems = (gsem0, gsem1, gsem2, gsem3)
    ssems = (ssem0, ssem1)

    def gather_desc(j, b):
        return pltpu.make_async_copy(
            tab_hbm.at[idx_v.at[j // _B0_W, lax.rem(j, _B0_W)]],
            rows_v.at[b], gsems[b])

    def store_descs(j, b):
        l = j // _B0_W
        b0 = wid * _B0_W + lax.rem(j, _B0_W)
        return [
            pltpu.make_async_copy(
                tr_v.at[b, pl.ds(8 * d0, 8)],
                out_hbm.at[l, d0, b0], ssems[b])
            for d0 in range(D // 8)
        ]

    def start_stores(j, b):
        for d_ in store_descs(j, b):
            d_.start()

    def wait_stores(j, b):
        for d_ in store_descs(j, b):
            d_.wait()

    lane = lax.iota(jnp.int32, 16)

    def transpose(gb, tb):
        src = rows_v.at[gb]
        dst = tr_v.at[tb]
        rowv = [lane + 16 * rb for rb in range(8)]

        def cbody(c, carry):
            t = jnp.bitwise_and(lane + c, 15)
            cols = [t + 16 * cb for cb in range(D // 16)]
            for cb in range(D // 16):
                for rb in range(8):
                    v = plsc.load_gather(src, [rowv[rb], cols[cb]])
                    plsc.store_scatter(dst, [cols[cb], rowv[rb]], v)
            return carry

        lax.fori_loop(0, 16, cbody, 0)

    def step(j, gb, tb):
        @pl.when(j >= 2)
        def _():
            wait_stores(j - 2, tb)

        gather_desc(j, gb).wait()
        transpose(gb, tb)

        @pl.when(j + 4 < _SLABS)
        def _():
            gather_desc(j + 4, gb).start()

        start_stores(j, tb)

    for g in range(4):
        gather_desc(g, g).start()

    def quad(p, carry):
        j = 4 * p
        step(j, 0, 0)
        step(j + 1, 1, 1)
        step(j + 2, 2, 0)
        step(j + 3, 3, 1)
        return carry

    lax.fori_loop(0, _SLABS // 4, quad, 0)

    wait_stores(_SLABS - 2, 0)
    wait_stores(_SLABS - 1, 1)


def kernel(term_ids, table):
    norm_tab = _normalize_table(table)
    idx3 = jnp.transpose(term_ids).reshape(L, _NB0, 128)
    phys = _gather_kernel(idx3, norm_tab)
    out = jnp.transpose(phys, (2, 4, 0, 1, 3)).reshape(B, L, D)
    return out


# speedup vs baseline: 1.9159x; 1.9159x over previous
import functools

import jax
import jax.numpy as jnp
from jax import lax
from jax.experimental import pallas as pl
from jax.experimental.pallas import tpu as pltpu
from jax.experimental.pallas import tpu_sc as plsc

N_TERMS = 100000
D = 64
B = 16384
L = 50
N_ROWS = B * L


_NORM_BLK = 2000


def _norm_body(x_ref, o_ref):
    x = x_ref[...]
    ss = jnp.sum(x * x, axis=1, keepdims=True)
    o_ref[...] = x * lax.rsqrt(jnp.maximum(ss, 1e-24))


def _normalize_table(table):
    return pl.pallas_call(
        _norm_body,
        grid=(N_TERMS // _NORM_BLK,),
        in_specs=[pl.BlockSpec((_NORM_BLK, D), lambda i: (i, 0))],
        out_specs=pl.BlockSpec((_NORM_BLK, D), lambda i: (i, 0)),
        out_shape=jax.ShapeDtypeStruct((N_TERMS, D), jnp.float32),
    )(table)


_NC = 2
_NS = 16
_NW = _NC * _NS
_NB0 = B // 128
_B0_W = _NB0 // _NW
_SLABS = L * _B0_W

_sc_mesh = plsc.VectorSubcoreMesh(core_axis_name="c", subcore_axis_name="s")


@functools.partial(
    pl.kernel,
    out_type=jax.ShapeDtypeStruct((L, D // 8, 128, 8, 128), jnp.float32),
    mesh=_sc_mesh,
    compiler_params=pltpu.CompilerParams(
        use_tc_tiling_on_sc=False, needs_layout_passes=False),
    scratch_types=[
        pltpu.VMEM((L, _B0_W, 128), jnp.int32),
        pltpu.VMEM((4, 128, D), jnp.float32),
        pltpu.VMEM((2, D, 128), jnp.float32),
        pltpu.SemaphoreType.DMA,
        pltpu.SemaphoreType.DMA,
        pltpu.SemaphoreType.DMA,
        pltpu.SemaphoreType.DMA,
        pltpu.SemaphoreType.DMA,
        pltpu.SemaphoreType.DMA,
    ],
)
def _gather_kernel(idx_hbm, tab_hbm, out_hbm, idx_v, rows_v, tr_v,
                   gsem0, gsem1, gsem2, gsem3, ssem0, ssem1):
    wid = lax.axis_index("s") * _NC + lax.axis_index("c")

    pltpu.sync_copy(idx_hbm.at[:, pl.ds(wid * _B0_W, _B0_W)], idx_v)

    gsems = (gsem0, gsem1, gsem2, gsem3)
    ssems = (ssem0, ssem1)

    def gather_desc(j, b):
        return pltpu.make_async_copy(
            tab_hbm.at[idx_v.at[j // _B0_W, lax.rem(j, _B0_W)]],
            rows_v.at[b], gsems[b])

    def store_descs(j, b):
        l = j // _B0_W
        b0 = wid * _B0_W + lax.rem(j, _B0_W)
        return [
            pltpu.make_async_copy(
                tr_v.at[b, pl.ds(8 * d0, 8)],
                out_hbm.at[l, d0, b0], ssems[b])
            for d0 in range(D // 8)
        ]

    def start_stores(j, b):
        for d_ in store_descs(j, b):
            d_.start()

    def wait_stores(j, b):
        for d_ in store_descs(j, b):
            d_.wait()

    lane = lax.iota(jnp.int32, 16)

    def transpose(gb, tb):
        src = rows_v.at[gb]
        dst = tr_v.at[tb]
        rowv = [lane + 16 * rb for rb in range(8)]

        def cbody(c, carry):
            t = jnp.bitwise_and(lane + c, 15)
            cols = [t + 16 * cb for cb in range(D // 16)]
            for cb in range(D // 16):
                vs = [plsc.load_gather(src, [rowv[rb], cols[cb]])
                      for rb in range(8)]
                for rb in range(8):
                    plsc.store_scatter(dst, [cols[cb], rowv[rb]], vs[rb])
            return carry

        lax.fori_loop(0, 16, cbody, 0)

    def step(j, gb, tb):
        @pl.when(j >= 2)
        def _():
            wait_stores(j - 2, tb)

        gather_desc(j, gb).wait()
        transpose(gb, tb)

        @pl.when(j + 4 < _SLABS)
        def _():
            gather_desc(j + 4, gb).start()

        start_stores(j, tb)

    for g in range(4):
        gather_desc(g, g).start()

    def quad(p, carry):
        j = 4 * p
        step(j, 0, 0)
        step(j + 1, 1, 1)
        step(j + 2, 2, 0)
        step(j + 3, 3, 1)
        return carry

    lax.fori_loop(0, _SLABS // 4, quad, 0)

    wait_stores(_SLABS - 2, 0)
    wait_stores(_SLABS - 1, 1)


def kernel(term_ids, table):
    norm_tab = _normalize_table(table)
    idx3 = jnp.transpose(term_ids).reshape(L, _NB0, 128)
    phys = _gather_kernel(idx3, norm_tab)
    out = jnp.transpose(phys, (2, 4, 0, 1, 3)).reshape(B, L, D)
    return out


# speedup vs baseline: 2.2757x; 1.1878x over previous
import functools

import jax
import jax.numpy as jnp
from jax import lax
from jax.experimental import pallas as pl
from jax.experimental.pallas import tpu as pltpu
from jax.experimental.pallas import tpu_sc as plsc

N_TERMS = 100000
D = 64
B = 16384
L = 50
N_ROWS = B * L


_NORM_BLK = 20000


def _norm_body(x_ref, o_ref):
    x = x_ref[...].reshape(_NORM_BLK // 2, 2 * D)
    sq = x * x
    sl = jnp.sum(sq[:, :D], axis=1, keepdims=True)
    sr = jnp.sum(sq[:, D:], axis=1, keepdims=True)
    cl = jnp.broadcast_to(lax.rsqrt(jnp.maximum(sl, 1e-24)), (_NORM_BLK // 2, D))
    cr = jnp.broadcast_to(lax.rsqrt(jnp.maximum(sr, 1e-24)), (_NORM_BLK // 2, D))
    xn = x * jnp.concatenate([cl, cr], axis=1)
    o_ref[...] = xn.reshape(_NORM_BLK * D)


def _normalize_table(table_flat):
    return pl.pallas_call(
        _norm_body,
        grid=(N_TERMS // _NORM_BLK,),
        in_specs=[pl.BlockSpec((_NORM_BLK * D,), lambda i: (i,))],
        out_specs=pl.BlockSpec((_NORM_BLK * D,), lambda i: (i,)),
        out_shape=jax.ShapeDtypeStruct((N_TERMS * D,), jnp.float32),
    )(table_flat)


_NC = 2
_NS = 16
_NW = _NC * _NS
_NB0 = B // 128
_B0_W = _NB0 // _NW
_SLABS = L * _B0_W

_sc_mesh = plsc.VectorSubcoreMesh(core_axis_name="c", subcore_axis_name="s")


@functools.partial(
    pl.kernel,
    out_type=jax.ShapeDtypeStruct((L, D // 8, 128, 8, 128), jnp.float32),
    mesh=_sc_mesh,
    compiler_params=pltpu.CompilerParams(
        use_tc_tiling_on_sc=False, needs_layout_passes=False),
    scratch_types=[
        pltpu.VMEM((L, _B0_W, 128), jnp.int32),
        pltpu.VMEM((4, 128, D), jnp.float32),
        pltpu.VMEM((2, D, 128), jnp.float32),
        pltpu.SemaphoreType.DMA,
        pltpu.SemaphoreType.DMA,
        pltpu.SemaphoreType.DMA,
        pltpu.SemaphoreType.DMA,
        pltpu.SemaphoreType.DMA,
        pltpu.SemaphoreType.DMA,
    ],
)
def _gather_kernel(idx_hbm, tab_hbm, out_hbm, idx_v, rows_v, tr_v,
                   gsem0, gsem1, gsem2, gsem3, ssem0, ssem1):
    wid = lax.axis_index("s") * _NC + lax.axis_index("c")

    pltpu.sync_copy(idx_hbm.at[:, pl.ds(wid * _B0_W, _B0_W)], idx_v)

    gsems = (gsem0, gsem1, gsem2, gsem3)
    ssems = (ssem0, ssem1)

    def gather_desc(j, b):
        return pltpu.make_async_copy(
            tab_hbm.at[idx_v.at[j // _B0_W, lax.rem(j, _B0_W)]],
            rows_v.at[b], gsems[b])

    def store_descs(j, b):
        l = j // _B0_W
        b0 = wid * _B0_W + lax.rem(j, _B0_W)
        return [
            pltpu.make_async_copy(
                tr_v.at[b, pl.ds(8 * d0, 8)],
                out_hbm.at[l, d0, b0], ssems[b])
            for d0 in range(D // 8)
        ]

    def start_stores(j, b):
        for d_ in store_descs(j, b):
            d_.start()

    def wait_stores(j, b):
        for d_ in store_descs(j, b):
            d_.wait()

    lane = lax.iota(jnp.int32, 16)

    def transpose(gb, tb):
        src = rows_v.at[gb]
        dst = tr_v.at[tb]
        rowv = [lane + 16 * rb for rb in range(8)]

        def cbody(c, carry):
            t = jnp.bitwise_and(lane + c, 15)
            cols = [t + 16 * cb for cb in range(D // 16)]
            for cb in range(D // 16):
                vs = [plsc.load_gather(src, [rowv[rb], cols[cb]])
                      for rb in range(8)]
                for rb in range(8):
                    plsc.store_scatter(dst, [cols[cb], rowv[rb]], vs[rb])
            return carry

        lax.fori_loop(0, 16, cbody, 0)

    def step(j, gb, tb):
        @pl.when(j >= 2)
        def _():
            wait_stores(j - 2, tb)

        gather_desc(j, gb).wait()
        transpose(gb, tb)

        @pl.when(j + 4 < _SLABS)
        def _():
            gather_desc(j + 4, gb).start()

        start_stores(j, tb)

    for g in range(4):
        gather_desc(g, g).start()

    def quad(p, carry):
        j = 4 * p
        step(j, 0, 0)
        step(j + 1, 1, 1)
        step(j + 2, 2, 0)
        step(j + 3, 3, 1)
        return carry

    lax.fori_loop(0, _SLABS // 4, quad, 0)

    wait_stores(_SLABS - 2, 0)
    wait_stores(_SLABS - 1, 1)


def kernel(term_ids, table):
    norm_tab = _normalize_table(table.reshape(N_TERMS * D)).reshape(N_TERMS, D)
    idx3 = jnp.transpose(term_ids).reshape(L, _NB0, 128)
    phys = _gather_kernel(idx3, norm_tab)
    out = jnp.transpose(phys, (2, 4, 0, 1, 3)).reshape(B, L, D)
    return out
